# baseline (speedup 1.0000x reference)
import functools

import jax
import jax.numpy as jnp
from jax import lax
from jax.experimental import pallas as pl
from jax.experimental.pallas import tpu as pltpu
from jax.experimental.pallas import tpu_sc as plsc

N = 10000
E = 320000
D = 128
DH = 64
G = 64
NCLS = 10

NC = 2
NS = 16
CHUNK = 128
CT = 160
EP = NS * CT * CHUNK
NBUF = 5
R = 10112
TS = R // NS
BLK = 632
NBLK = R // BLK

_MESH = plsc.VectorSubcoreMesh(core_axis_name="c", subcore_axis_name="s")
_SC_PARAMS = pltpu.CompilerParams(use_tc_tiling_on_sc=False)


def _deg_body(dstI, zr, ones_hbm, d_out, idx_d, ones_v, acc):
    c = lax.axis_index("c")
    t = lax.axis_index("s")
    half = CT // 2
    pltpu.sync_copy(zr, acc.at[pl.ds(t * TS, TS)])
    pltpu.sync_copy(dstI.at[c, t], idx_d)
    pltpu.sync_copy(ones_hbm, ones_v)
    plsc.subcore_barrier()

    def body(j, carry):
        pltpu.sync_copy(ones_v, acc.at[idx_d.at[j]], add=True)
        return carry

    lax.fori_loop(0, half, body, 0)
    plsc.subcore_barrier()
    pltpu.sync_copy(acc.at[pl.ds(t * TS, TS)], d_out.at[c, pl.ds(t * TS, TS)])


_deg_call = functools.partial(
    pl.kernel,
    mesh=_MESH,
    out_type=jax.ShapeDtypeStruct((NC, R, 16), jnp.float32),
    compiler_params=_SC_PARAMS,
    scratch_types=[
        pltpu.VMEM((CT // 2, CHUNK), jnp.int32),
        pltpu.VMEM((CHUNK, 16), jnp.float32),
        pltpu.VMEM_SHARED((R, 16), jnp.float32),
    ],
)(_deg_body)


def _edge_body(hd, srcI, dstI, zr, s_out, idx_s, idx_d, rows, acc, *sems):
    gsem = sems[:NBUF]
    c = lax.axis_index("c")
    t = lax.axis_index("s")
    pltpu.sync_copy(zr, acc.at[pl.ds(t * TS, TS)])
    pltpu.sync_copy(srcI.at[t], idx_s)
    pltpu.sync_copy(dstI.at[t], idx_d)

    plsc.subcore_barrier()
    table = hd.at[c]
    for j in range(NBUF):
        pltpu.async_copy(table.at[idx_s.at[j]], rows.at[j], gsem[j])

    def outer(i, carry):
        j0 = i * NBUF
        for b in range(NBUF):
            j = j0 + b
            pltpu.make_async_copy(table.at[idx_s.at[j]], rows.at[b],
                                  gsem[b]).wait()
            pltpu.sync_copy(rows.at[b], acc.at[idx_d.at[j]], add=True)

            @pl.when(j + NBUF < CT)
            def _():
                pltpu.async_copy(table.at[idx_s.at[j + NBUF]], rows.at[b],
                                 gsem[b])

        return carry

    lax.fori_loop(0, CT // NBUF, outer, 0)
    plsc.subcore_barrier()
    pltpu.sync_copy(acc.at[pl.ds(t * TS, TS)], s_out.at[c, pl.ds(t * TS, TS)])


_edge_call = functools.partial(
    pl.kernel,
    mesh=_MESH,
    out_type=jax.ShapeDtypeStruct((NC, R, DH), jnp.float32),
    compiler_params=_SC_PARAMS,
    scratch_types=[
        pltpu.VMEM((CT, CHUNK), jnp.int32),
        pltpu.VMEM((CT, CHUNK), jnp.int32),
        pltpu.VMEM((NBUF, CHUNK, DH), jnp.float32),
        pltpu.VMEM_SHARED((R, DH), jnp.float32),
    ] + [pltpu.SemaphoreType.DMA] * NBUF,
)(_edge_body)


def _head_body(x_ref, w_ref, dinv_ref, o_ref):
    h = jnp.dot(x_ref[...], w_ref[...], preferred_element_type=jnp.float32)
    hd = h * dinv_ref[...]
    o_ref[0] = hd[:, :DH]
    o_ref[1] = hd[:, DH:]


def _head(xp, W1, dinv):
    return pl.pallas_call(
        _head_body,
        grid=(NBLK,),
        in_specs=[
            pl.BlockSpec((BLK, D), lambda i: (i, 0)),
            pl.BlockSpec((D, D), lambda i: (0, 0)),
            pl.BlockSpec((BLK, 1), lambda i: (i, 0)),
        ],
        out_specs=pl.BlockSpec((NC, BLK, DH), lambda i: (0, i, 0)),
        out_shape=jax.ShapeDtypeStruct((NC, R, DH), jnp.float32),
    )(xp, W1, dinv)


def _mid_body(s_ref, hd_ref, dinv_ref, b_ref, w_ref, o_ref):
    t = jnp.concatenate([s_ref[0] + hd_ref[0], s_ref[1] + hd_ref[1]], axis=1)
    a = jnp.maximum(t * dinv_ref[...] + b_ref[...], 0.0)
    hd2 = jnp.dot(a, w_ref[...],
                  preferred_element_type=jnp.float32) * dinv_ref[...]
    o_ref[0] = hd2[:, :DH]
    o_ref[1] = hd2[:, DH:]


def _mid(s, hd, dinv, b, W2):
    return pl.pallas_call(
        _mid_body,
        grid=(NBLK,),
        in_specs=[
            pl.BlockSpec((NC, BLK, DH), lambda i: (0, i, 0)),
            pl.BlockSpec((NC, BLK, DH), lambda i: (0, i, 0)),
            pl.BlockSpec((BLK, 1), lambda i: (i, 0)),
            pl.BlockSpec((1, D), lambda i: (0, 0)),
            pl.BlockSpec((D, D), lambda i: (0, 0)),
        ],
        out_specs=pl.BlockSpec((NC, BLK, DH), lambda i: (0, i, 0)),
        out_shape=jax.ShapeDtypeStruct((NC, R, DH), jnp.float32),
    )(s, hd, dinv, b, W2)


def _tail_body(s_ref, hd_ref, dinv_ref, b_ref, batch_ref, wl_ref, bl_ref,
               o_ref, sums, cnts):
    i = pl.program_id(0)

    @pl.when(i == 0)
    def _():
        sums[...] = jnp.zeros_like(sums)
        cnts[...] = jnp.zeros_like(cnts)

    t = jnp.concatenate([s_ref[0] + hd_ref[0], s_ref[1] + hd_ref[1]], axis=1)
    h2 = jnp.maximum(t * dinv_ref[...] + b_ref[...], 0.0)
    onehot = (batch_ref[...] ==
              lax.broadcasted_iota(jnp.int32, (BLK, G), 1)).astype(jnp.float32)
    sums[...] += lax.dot_general(onehot, h2, (((0,), (0,)), ((), ())),
                                 preferred_element_type=jnp.float32)
    cnts[...] += lax.dot_general(onehot, jnp.ones((BLK, D), jnp.float32),
                                 (((0,), (0,)), ((), ())),
                                 preferred_element_type=jnp.float32)

    @pl.when(i == NBLK - 1)
    def _():
        pooled = sums[...] / jnp.maximum(cnts[...], 1.0)
        o_ref[...] = jnp.dot(pooled, wl_ref[...],
                             preferred_element_type=jnp.float32) + bl_ref[...]


def _tail(s, hd, dinv, b, batchp, Wlp, blp):
    return pl.pallas_call(
        _tail_body,
        grid=(NBLK,),
        in_specs=[
            pl.BlockSpec((NC, BLK, DH), lambda i: (0, i, 0)),
            pl.BlockSpec((NC, BLK, DH), lambda i: (0, i, 0)),
            pl.BlockSpec((BLK, 1), lambda i: (i, 0)),
            pl.BlockSpec((1, D), lambda i: (0, 0)),
            pl.BlockSpec((BLK, 1), lambda i: (i, 0)),
            pl.BlockSpec((D, 16), lambda i: (0, 0)),
            pl.BlockSpec((1, 16), lambda i: (0, 0)),
        ],
        out_specs=pl.BlockSpec((G, 16), lambda i: (0, 0)),
        out_shape=jax.ShapeDtypeStruct((G, 16), jnp.float32),
        scratch_shapes=[
            pltpu.VMEM((G, D), jnp.float32),
            pltpu.VMEM((G, D), jnp.float32),
        ],
    )(s, hd, dinv, b, batchp, Wlp, blp)


def kernel(x, edge_index, batch, W1, b1, W2, b2, Wl, bl):
    src = edge_index[0].astype(jnp.int32)
    dst = edge_index[1].astype(jnp.int32)
    pad = EP - E
    src_pad = (jnp.arange(pad, dtype=jnp.int32) * 997) % N
    src3 = jnp.concatenate([src, src_pad]).reshape(NS, CT, CHUNK)
    dst_pad = N + (jnp.arange(pad, dtype=jnp.int32) % (R - N))
    dst3 = jnp.concatenate([dst, dst_pad]).reshape(NS, CT, CHUNK)
    xp = jnp.pad(x, ((0, R - N), (0, 0)))
    batchp = jnp.pad(batch.astype(jnp.int32), (0, R - N),
                     constant_values=G).reshape(R, 1)
    z16 = jnp.zeros((TS, 16), jnp.float32)
    zDH = jnp.zeros((TS, DH), jnp.float32)
    ones_rows = jnp.concatenate(
        [jnp.ones((CHUNK, 1), jnp.float32),
         jnp.zeros((CHUNK, 15), jnp.float32)], axis=1)
    Wlp = jnp.pad(Wl, ((0, 0), (0, 16 - NCLS)))
    blp = jnp.pad(bl, (0, 16 - NCLS)).reshape(1, 16)
    b1r = b1.reshape(1, D)
    b2r = b2.reshape(1, D)

    dst4 = dst3.reshape(NC, NS, CT // 2, CHUNK)
    deg = _deg_call(dst4, z16, ones_rows)
    cnt = deg[0, :, 0] + deg[1, :, 0] + 1.0
    dinv = lax.rsqrt(cnt).reshape(R, 1)

    hd1 = _head(xp, W1, dinv)
    s1 = _edge_call(hd1, src3, dst3, zDH)
    hd2 = _mid(s1, hd1, dinv, b1r, W2)
    s2 = _edge_call(hd2, src3, dst3, zDH)
    out16 = _tail(s2, hd2, dinv, b2r, batchp, Wlp, blp)
    return out16[:, :NCLS]


# speedup vs baseline: 1.0714x; 1.0714x over previous
import functools

import jax
import jax.numpy as jnp
from jax import lax
from jax.experimental import pallas as pl
from jax.experimental.pallas import tpu as pltpu
from jax.experimental.pallas import tpu_sc as plsc

N = 10000
E = 320000
D = 128
DH = 64
G = 64
NCLS = 10

NC = 2
NS = 16
CHUNK = 128
CT = 160
HF = CT // 2
EP = NS * CT * CHUNK
R = 10112
R2 = R // 2
TS = R // NS
BLK = 1264
BLK2 = BLK // 2
NBLK = R // BLK
NBUF = 4

_MESH = plsc.VectorSubcoreMesh(core_axis_name="c", subcore_axis_name="s")
_SC_PARAMS = pltpu.CompilerParams(use_tc_tiling_on_sc=False,
                                  needs_layout_passes=False)


def _deg_body(eiI, zr, ones_hbm, d_out, idx_sd, ones_v, ext, out1d, acc):
    c = lax.axis_index("c")
    t = lax.axis_index("s")
    pltpu.sync_copy(zr, acc.at[pl.ds(t * TS, TS)])
    pltpu.sync_copy(eiI.at[t, pl.ds(c * HF, HF)], idx_sd)
    pltpu.sync_copy(ones_hbm, ones_v)
    plsc.subcore_barrier()

    def body(j, carry):
        pltpu.sync_copy(ones_v, acc.at[idx_sd.at[j, 1]], add=True)
        return carry

    lax.fori_loop(0, HF, body, 0)
    plsc.subcore_barrier()

    pltpu.sync_copy(acc.at[pl.ds(t * TS, TS)], ext)
    lane = lax.iota(jnp.int32, 16)
    zero16 = jnp.zeros((16,), jnp.int32)

    def extract(i, carry):
        vals = plsc.load_gather(ext, [lane + i * 16, zero16])
        out1d[pl.ds(i * 16, 16)] = vals
        return carry

    lax.fori_loop(0, TS // 16, extract, 0)
    pltpu.sync_copy(out1d, d_out.at[c, t])


_deg_call = functools.partial(
    pl.kernel,
    mesh=_MESH,
    out_type=jax.ShapeDtypeStruct((NC, NS, TS), jnp.float32),
    compiler_params=_SC_PARAMS,
    scratch_types=[
        pltpu.VMEM((HF, 2, CHUNK), jnp.int32),
        pltpu.VMEM((CHUNK, 16), jnp.float32),
        pltpu.VMEM((TS, 16), jnp.float32),
        pltpu.VMEM((TS,), jnp.float32),
        pltpu.VMEM_SHARED((R, 16), jnp.float32),
    ],
)(_deg_body)


def _edge_body(hd, eiI, zr, s_out, idx_sd, rows, acc, *gsem):
    c = lax.axis_index("c")
    t = lax.axis_index("s")
    pltpu.sync_copy(zr, acc.at[pl.ds(t * TS, TS)])
    pltpu.sync_copy(eiI.at[t], idx_sd)

    plsc.subcore_barrier()
    table = hd.at[c]
    for j in range(NBUF):
        pltpu.async_copy(table.at[idx_sd.at[j, 0]], rows.at[j], gsem[j])

    def outer(i, carry):
        j0 = i * NBUF
        for b in range(NBUF):
            j = j0 + b
            pltpu.make_async_copy(table.at[idx_sd.at[j, 0]], rows.at[b],
                                  gsem[b]).wait()
            pltpu.sync_copy(rows.at[b], acc.at[idx_sd.at[j, 1]], add=True)

            @pl.when(j + NBUF < CT)
            def _():
                pltpu.async_copy(table.at[idx_sd.at[j + NBUF, 0]], rows.at[b],
                                 gsem[b])

        return carry

    lax.fori_loop(0, CT // NBUF, outer, 0)
    plsc.subcore_barrier()
    pltpu.sync_copy(acc.at[pl.ds(t * TS, TS)], s_out.at[c, pl.ds(t * TS, TS)])


_edge_call = functools.partial(
    pl.kernel,
    mesh=_MESH,
    out_type=jax.ShapeDtypeStruct((NC, R, DH), jnp.float32),
    compiler_params=_SC_PARAMS,
    scratch_types=[
        pltpu.VMEM((CT, 2, CHUNK), jnp.int32),
        pltpu.VMEM((NBUF, CHUNK, DH), jnp.float32),
        pltpu.VMEM_SHARED((R, DH), jnp.float32),
    ] + [pltpu.SemaphoreType.DMA] * NBUF,
)(_edge_body)


def _pack(half):
    return jnp.reshape(half, (BLK2, D))


def _unpack(plane):
    return jnp.reshape(plane, (BLK, DH))


def _head_body(x_ref, w_ref, dinv_ref, o_ref):
    h = jnp.dot(x_ref[...], w_ref[...], preferred_element_type=jnp.float32)
    hd = h * dinv_ref[...]
    o_ref[0] = hd[:, :DH]
    o_ref[1] = hd[:, DH:]


def _head(xp, W1, dinv):
    return pl.pallas_call(
        _head_body,
        grid=(NBLK,),
        in_specs=[
            pl.BlockSpec((BLK, D), lambda i: (i, 0)),
            pl.BlockSpec((D, D), lambda i: (0, 0)),
            pl.BlockSpec((BLK, 1), lambda i: (i, 0)),
        ],
        out_specs=pl.BlockSpec((NC, BLK, DH), lambda i: (0, i, 0)),
        out_shape=jax.ShapeDtypeStruct((NC, R, DH), jnp.float32),
    )(xp, W1, dinv)


def _mid_body(s_ref, hd_ref, dinv_ref, b_ref, w_ref, o_ref):
    t = jnp.concatenate([s_ref[0] + hd_ref[0], s_ref[1] + hd_ref[1]], axis=1)
    a = jnp.maximum(t * dinv_ref[...] + b_ref[...], 0.0)
    hd2 = jnp.dot(a, w_ref[...],
                  preferred_element_type=jnp.float32) * dinv_ref[...]
    o_ref[0] = hd2[:, :DH]
    o_ref[1] = hd2[:, DH:]


def _mid(s, hd, dinv, b, W2):
    return pl.pallas_call(
        _mid_body,
        grid=(NBLK,),
        in_specs=[
            pl.BlockSpec((NC, BLK, DH), lambda i: (0, i, 0)),
            pl.BlockSpec((NC, BLK, DH), lambda i: (0, i, 0)),
            pl.BlockSpec((BLK, 1), lambda i: (i, 0)),
            pl.BlockSpec((1, D), lambda i: (0, 0)),
            pl.BlockSpec((D, D), lambda i: (0, 0)),
        ],
        out_specs=pl.BlockSpec((NC, BLK, DH), lambda i: (0, i, 0)),
        out_shape=jax.ShapeDtypeStruct((NC, R, DH), jnp.float32),
    )(s, hd, dinv, b, W2)


def _tail_body(s_ref, hd_ref, dinv_ref, b_ref, batch_ref, wl_ref, bl_ref,
               o_ref, sums, cnts):
    i = pl.program_id(0)

    @pl.when(i == 0)
    def _():
        sums[...] = jnp.zeros_like(sums)
        cnts[...] = jnp.zeros_like(cnts)

    t = jnp.concatenate([s_ref[0] + hd_ref[0], s_ref[1] + hd_ref[1]], axis=1)
    h2 = jnp.maximum(t * dinv_ref[...] + b_ref[...], 0.0)
    onehot = (batch_ref[...] ==
              lax.broadcasted_iota(jnp.int32, (BLK, G), 1)).astype(jnp.float32)
    sums[...] += lax.dot_general(onehot, h2, (((0,), (0,)), ((), ())),
                                 preferred_element_type=jnp.float32)
    cnts[...] += lax.dot_general(onehot, jnp.ones((BLK, D), jnp.float32),
                                 (((0,), (0,)), ((), ())),
                                 preferred_element_type=jnp.float32)

    @pl.when(i == NBLK - 1)
    def _():
        pooled = sums[...] / jnp.maximum(cnts[...], 1.0)
        o_ref[...] = jnp.dot(pooled, wl_ref[...],
                             preferred_element_type=jnp.float32) + bl_ref[...]


def _tail(s, hd, dinv, b, batchp, Wlp, blp):
    return pl.pallas_call(
        _tail_body,
        grid=(NBLK,),
        in_specs=[
            pl.BlockSpec((NC, BLK, DH), lambda i: (0, i, 0)),
            pl.BlockSpec((NC, BLK, DH), lambda i: (0, i, 0)),
            pl.BlockSpec((BLK, 1), lambda i: (i, 0)),
            pl.BlockSpec((1, D), lambda i: (0, 0)),
            pl.BlockSpec((BLK, 1), lambda i: (i, 0)),
            pl.BlockSpec((D, 16), lambda i: (0, 0)),
            pl.BlockSpec((1, 16), lambda i: (0, 0)),
        ],
        out_specs=pl.BlockSpec((G, 16), lambda i: (0, 0)),
        out_shape=jax.ShapeDtypeStruct((G, 16), jnp.float32),
        scratch_shapes=[
            pltpu.VMEM((G, D), jnp.float32),
            pltpu.VMEM((G, D), jnp.float32),
        ],
    )(s, hd, dinv, b, batchp, Wlp, blp)


def kernel(x, edge_index, batch, W1, b1, W2, b2, Wl, bl):
    nchunk = E // CHUNK
    padc = NS * CT - nchunk
    ei = edge_index.astype(jnp.int32).reshape(2, nchunk, CHUNK)
    ei = jnp.transpose(ei, (1, 0, 2))
    pad_s = ((jnp.arange(padc * CHUNK, dtype=jnp.int32) * 997) % N).reshape(
        padc, 1, CHUNK)
    pad_d = (N + (jnp.arange(padc * CHUNK, dtype=jnp.int32) % (R - N))
             ).reshape(padc, 1, CHUNK)
    ei4 = jnp.concatenate([ei, jnp.concatenate([pad_s, pad_d], axis=1)],
                          axis=0).reshape(NS, CT, 2, CHUNK)

    xp = jnp.pad(x, ((0, R - N), (0, 0)))
    batchp = jnp.pad(batch.astype(jnp.int32), (0, R - N),
                     constant_values=G).reshape(R, 1)
    z16 = jnp.zeros((TS, 16), jnp.float32)
    zDH = jnp.zeros((TS, DH), jnp.float32)
    ones_rows = jnp.concatenate(
        [jnp.ones((CHUNK, 1), jnp.float32),
         jnp.zeros((CHUNK, 15), jnp.float32)], axis=1)
    Wlp = jnp.pad(Wl, ((0, 0), (0, 16 - NCLS)))
    blp = jnp.pad(bl, (0, 16 - NCLS)).reshape(1, 16)
    b1r = b1.reshape(1, D)
    b2r = b2.reshape(1, D)

    deg = _deg_call(ei4, z16, ones_rows)
    cnt = deg.reshape(NC, R)
    dinv = lax.rsqrt(cnt[0] + cnt[1] + 1.0).reshape(R, 1)

    hd1 = _head(xp, W1, dinv)
    s1 = _edge_call(hd1, ei4, zDH)
    hd2 = _mid(s1, hd1, dinv, b1r, W2)
    s2 = _edge_call(hd2, ei4, zDH)
    out16 = _tail(s2, hd2, dinv, b2r, batchp, Wlp, blp)
    return out16[:, :NCLS]


# speedup vs baseline: 1.2011x; 1.1210x over previous
import functools

import jax
import jax.numpy as jnp
from jax import lax
from jax.experimental import pallas as pl
from jax.experimental.pallas import tpu as pltpu
from jax.experimental.pallas import tpu_sc as plsc

N = 10000
E = 320000
D = 128
DH = 64
G = 64
NCLS = 10

NC = 2
NS = 16
CHUNK = 128
CT = 160
HF = CT // 2
EP = NS * CT * CHUNK
R = 10112
R2 = R // 2
TS = R // NS
BLK = 1264
BLK2 = BLK // 2
NBLK = R // BLK
NBUF = 4

_MESH = plsc.VectorSubcoreMesh(core_axis_name="c", subcore_axis_name="s")
_SC_PARAMS = pltpu.CompilerParams(use_tc_tiling_on_sc=False,
                                  needs_layout_passes=False)


def _deg_body(eiI, zr, ones_hbm, d_out, idx_sd, ones_v, ext, out1d, acc):
    c = lax.axis_index("c")
    t = lax.axis_index("s")
    pltpu.sync_copy(zr, acc.at[pl.ds(t * TS, TS)])
    pltpu.sync_copy(eiI.at[t, pl.ds(c * HF, HF)], idx_sd)
    pltpu.sync_copy(ones_hbm, ones_v)
    plsc.subcore_barrier()

    def body(j, carry):
        pltpu.sync_copy(ones_v, acc.at[idx_sd.at[j, 1]], add=True)
        return carry

    lax.fori_loop(0, HF, body, 0)
    plsc.subcore_barrier()

    pltpu.sync_copy(acc.at[pl.ds(t * TS, TS)], ext)
    lane = lax.iota(jnp.int32, 16)
    zero16 = jnp.zeros((16,), jnp.int32)

    def extract(i, carry):
        vals = plsc.load_gather(ext, [lane + i * 16, zero16])
        out1d[pl.ds(i * 16, 16)] = vals
        return carry

    lax.fori_loop(0, TS // 16, extract, 0)
    pltpu.sync_copy(out1d, d_out.at[c, t])


_deg_call = functools.partial(
    pl.kernel,
    mesh=_MESH,
    out_type=jax.ShapeDtypeStruct((NC, NS, TS), jnp.float32),
    compiler_params=_SC_PARAMS,
    scratch_types=[
        pltpu.VMEM((HF, 2, CHUNK), jnp.int32),
        pltpu.VMEM((CHUNK, 16), jnp.float32),
        pltpu.VMEM((TS, 16), jnp.float32),
        pltpu.VMEM((TS,), jnp.float32),
        pltpu.VMEM_SHARED((R, 16), jnp.float32),
    ],
)(_deg_body)


def _edge_body(hd, eiI, zr, s_out, idx_sd, rows, acc, *gsem):
    c = lax.axis_index("c")
    t = lax.axis_index("s")
    pltpu.sync_copy(zr, acc.at[pl.ds(t * TS, TS)])
    pltpu.sync_copy(eiI.at[t], idx_sd)

    plsc.subcore_barrier()
    table = hd.at[c]
    for j in range(NBUF):
        pltpu.async_copy(table.at[idx_sd.at[j, 0]], rows.at[j], gsem[j])

    def outer(i, carry):
        j0 = i * NBUF
        for b in range(NBUF):
            j = j0 + b
            pltpu.make_async_copy(table.at[idx_sd.at[j, 0]], rows.at[b],
                                  gsem[b]).wait()
            pltpu.sync_copy(rows.at[b], acc.at[idx_sd.at[j, 1]], add=True)

            @pl.when(j + NBUF < CT)
            def _():
                pltpu.async_copy(table.at[idx_sd.at[j + NBUF, 0]], rows.at[b],
                                 gsem[b])

        return carry

    lax.fori_loop(0, CT // NBUF, outer, 0)
    plsc.subcore_barrier()
    pltpu.sync_copy(acc.at[pl.ds(t * TS, TS)], s_out.at[c, pl.ds(t * TS, TS)])


_edge_call = functools.partial(
    pl.kernel,
    mesh=_MESH,
    out_type=jax.ShapeDtypeStruct((NC, R, DH), jnp.float32),
    compiler_params=_SC_PARAMS,
    scratch_types=[
        pltpu.VMEM((CT, 2, CHUNK), jnp.int32),
        pltpu.VMEM((NBUF, CHUNK, DH), jnp.float32),
        pltpu.VMEM_SHARED((R, DH), jnp.float32),
    ] + [pltpu.SemaphoreType.DMA] * NBUF,
)(_edge_body)


def _pack2(hd):
    h3 = jnp.reshape(hd, (BLK2, 2, D))
    p0 = jnp.concatenate([h3[:, 0, :DH], h3[:, 1, :DH]], axis=1)
    p1 = jnp.concatenate([h3[:, 0, DH:], h3[:, 1, DH:]], axis=1)
    return p0, p1


def _unpack2(p0, p1):
    u = jnp.concatenate([p0[:, :DH], p1[:, :DH]], axis=1)[:, None, :]
    v = jnp.concatenate([p0[:, DH:], p1[:, DH:]], axis=1)[:, None, :]
    return jnp.reshape(jnp.concatenate([u, v], axis=1), (BLK, D))


def _head_body(x_ref, w_ref, dinv_ref, o_ref):
    h = jnp.dot(x_ref[...], w_ref[...], preferred_element_type=jnp.float32)
    hd = h * dinv_ref[...]
    p0, p1 = _pack2(hd)
    o_ref[0] = p0
    o_ref[1] = p1


def _head(xp, W1, dinv):
    return pl.pallas_call(
        _head_body,
        grid=(NBLK,),
        in_specs=[
            pl.BlockSpec((BLK, D), lambda i: (i, 0)),
            pl.BlockSpec((D, D), lambda i: (0, 0)),
            pl.BlockSpec((BLK, 1), lambda i: (i, 0)),
        ],
        out_specs=pl.BlockSpec((NC, BLK2, D), lambda i: (0, i, 0)),
        out_shape=jax.ShapeDtypeStruct((NC, R2, D), jnp.float32),
    )(xp, W1, dinv)


def _mid_body(s_ref, hd_ref, dinv_ref, b_ref, w_ref, o_ref):
    t = _unpack2(s_ref[0] + hd_ref[0], s_ref[1] + hd_ref[1])
    a = jnp.maximum(t * dinv_ref[...] + b_ref[...], 0.0)
    hd2 = jnp.dot(a, w_ref[...],
                  preferred_element_type=jnp.float32) * dinv_ref[...]
    p0, p1 = _pack2(hd2)
    o_ref[0] = p0
    o_ref[1] = p1


def _mid(s, hd, dinv, b, W2):
    return pl.pallas_call(
        _mid_body,
        grid=(NBLK,),
        in_specs=[
            pl.BlockSpec((NC, BLK2, D), lambda i: (0, i, 0)),
            pl.BlockSpec((NC, BLK2, D), lambda i: (0, i, 0)),
            pl.BlockSpec((BLK, 1), lambda i: (i, 0)),
            pl.BlockSpec((1, D), lambda i: (0, 0)),
            pl.BlockSpec((D, D), lambda i: (0, 0)),
        ],
        out_specs=pl.BlockSpec((NC, BLK2, D), lambda i: (0, i, 0)),
        out_shape=jax.ShapeDtypeStruct((NC, R2, D), jnp.float32),
    )(s, hd, dinv, b, W2)


def _tail_body(s_ref, hd_ref, dinv_ref, b_ref, batch_ref, wl_ref, bl_ref,
               o_ref, sums, cnts):
    i = pl.program_id(0)

    @pl.when(i == 0)
    def _():
        sums[...] = jnp.zeros_like(sums)
        cnts[...] = jnp.zeros_like(cnts)

    t = _unpack2(s_ref[0] + hd_ref[0], s_ref[1] + hd_ref[1])
    h2 = jnp.maximum(t * dinv_ref[...] + b_ref[...], 0.0)
    onehot = (batch_ref[...] ==
              lax.broadcasted_iota(jnp.int32, (BLK, G), 1)).astype(jnp.float32)
    sums[...] += lax.dot_general(onehot, h2, (((0,), (0,)), ((), ())),
                                 preferred_element_type=jnp.float32)
    cnts[...] += lax.dot_general(onehot, jnp.ones((BLK, D), jnp.float32),
                                 (((0,), (0,)), ((), ())),
                                 preferred_element_type=jnp.float32)

    @pl.when(i == NBLK - 1)
    def _():
        pooled = sums[...] / jnp.maximum(cnts[...], 1.0)
        o_ref[...] = jnp.dot(pooled, wl_ref[...],
                             preferred_element_type=jnp.float32) + bl_ref[...]


def _tail(s, hd, dinv, b, batchp, Wlp, blp):
    return pl.pallas_call(
        _tail_body,
        grid=(NBLK,),
        in_specs=[
            pl.BlockSpec((NC, BLK2, D), lambda i: (0, i, 0)),
            pl.BlockSpec((NC, BLK2, D), lambda i: (0, i, 0)),
            pl.BlockSpec((BLK, 1), lambda i: (i, 0)),
            pl.BlockSpec((1, D), lambda i: (0, 0)),
            pl.BlockSpec((BLK, 1), lambda i: (i, 0)),
            pl.BlockSpec((D, 16), lambda i: (0, 0)),
            pl.BlockSpec((1, 16), lambda i: (0, 0)),
        ],
        out_specs=pl.BlockSpec((G, 16), lambda i: (0, 0)),
        out_shape=jax.ShapeDtypeStruct((G, 16), jnp.float32),
        scratch_shapes=[
            pltpu.VMEM((G, D), jnp.float32),
            pltpu.VMEM((G, D), jnp.float32),
        ],
    )(s, hd, dinv, b, batchp, Wlp, blp)


def kernel(x, edge_index, batch, W1, b1, W2, b2, Wl, bl):
    nchunk = E // CHUNK
    padc = NS * CT - nchunk
    ei = edge_index.astype(jnp.int32).reshape(2, nchunk, CHUNK)
    ei = jnp.transpose(ei, (1, 0, 2))
    pad_s = ((jnp.arange(padc * CHUNK, dtype=jnp.int32) * 997) % N).reshape(
        padc, 1, CHUNK)
    pad_d = (N + (jnp.arange(padc * CHUNK, dtype=jnp.int32) % (R - N))
             ).reshape(padc, 1, CHUNK)
    ei4 = jnp.concatenate([ei, jnp.concatenate([pad_s, pad_d], axis=1)],
                          axis=0).reshape(NS, CT, 2, CHUNK)

    xp = jnp.pad(x, ((0, R - N), (0, 0)))
    batchp = jnp.pad(batch.astype(jnp.int32), (0, R - N),
                     constant_values=G).reshape(R, 1)
    z16 = jnp.zeros((TS, 16), jnp.float32)
    zDH = jnp.zeros((TS, DH), jnp.float32)
    ones_rows = jnp.concatenate(
        [jnp.ones((CHUNK, 1), jnp.float32),
         jnp.zeros((CHUNK, 15), jnp.float32)], axis=1)
    Wlp = jnp.pad(Wl, ((0, 0), (0, 16 - NCLS)))
    blp = jnp.pad(bl, (0, 16 - NCLS)).reshape(1, 16)
    b1r = b1.reshape(1, D)
    b2r = b2.reshape(1, D)

    deg = _deg_call(ei4, z16, ones_rows)
    cnt = deg.reshape(NC, R)
    dinv = lax.rsqrt(cnt[0] + cnt[1] + 1.0).reshape(R, 1)

    hd1 = _head(xp, W1, dinv)
    s1 = _edge_call(hd1.reshape(NC, R, DH), ei4, zDH)
    hd2 = _mid(s1.reshape(NC, R2, D), hd1, dinv, b1r, W2)
    s2 = _edge_call(hd2.reshape(NC, R, DH), ei4, zDH)
    out16 = _tail(s2.reshape(NC, R2, D), hd2, dinv, b2r, batchp, Wlp, blp)
    return out16[:, :NCLS]


# speedup vs baseline: 1.2140x; 1.0107x over previous
import functools

import jax
import jax.numpy as jnp
from jax import lax
from jax.experimental import pallas as pl
from jax.experimental.pallas import tpu as pltpu
from jax.experimental.pallas import tpu_sc as plsc

N = 10000
E = 320000
D = 128
DH = 64
G = 64
NCLS = 10

NC = 2
NS = 16
CHUNK = 128
CT = 160
HF = CT // 2
EP = NS * CT * CHUNK
R = 10112
R2 = R // 2
TS = R // NS
BLK = 2528
BLK2 = BLK // 2
NBLK = R // BLK
NBUF = 4

_MESH = plsc.VectorSubcoreMesh(core_axis_name="c", subcore_axis_name="s")
_SC_PARAMS = pltpu.CompilerParams(use_tc_tiling_on_sc=False,
                                  needs_layout_passes=False)


def _deg_body(eiI, zr, ones_hbm, d_out, idx_sd, ones_v, ext, out1d, acc):
    c = lax.axis_index("c")
    t = lax.axis_index("s")
    pltpu.sync_copy(zr, acc.at[pl.ds(t * TS, TS)])
    pltpu.sync_copy(eiI.at[t, pl.ds(c * HF, HF)], idx_sd)
    pltpu.sync_copy(ones_hbm, ones_v)
    plsc.subcore_barrier()

    def body(j, carry):
        pltpu.sync_copy(ones_v, acc.at[idx_sd.at[j, 1]], add=True)
        return carry

    lax.fori_loop(0, HF, body, 0)
    plsc.subcore_barrier()

    pltpu.sync_copy(acc.at[pl.ds(t * TS, TS)], ext)
    lane = lax.iota(jnp.int32, 16)
    zero16 = jnp.zeros((16,), jnp.int32)

    def extract(i, carry):
        vals = plsc.load_gather(ext, [lane + i * 16, zero16])
        out1d[pl.ds(i * 16, 16)] = vals
        return carry

    lax.fori_loop(0, TS // 16, extract, 0)
    pltpu.sync_copy(out1d, d_out.at[c, t])


_deg_call = functools.partial(
    pl.kernel,
    mesh=_MESH,
    out_type=jax.ShapeDtypeStruct((NC, NS, TS), jnp.float32),
    compiler_params=_SC_PARAMS,
    scratch_types=[
        pltpu.VMEM((HF, 2, CHUNK), jnp.int32),
        pltpu.VMEM((CHUNK, 16), jnp.float32),
        pltpu.VMEM((TS, 16), jnp.float32),
        pltpu.VMEM((TS,), jnp.float32),
        pltpu.VMEM_SHARED((R, 16), jnp.float32),
    ],
)(_deg_body)


def _edge_body(hd, eiI, zr, s_out, idx_sd, rows, acc, *gsem):
    c = lax.axis_index("c")
    t = lax.axis_index("s")
    pltpu.sync_copy(zr, acc.at[pl.ds(t * TS, TS)])
    pltpu.sync_copy(eiI.at[t], idx_sd)

    plsc.subcore_barrier()
    table = hd.at[c]
    for j in range(NBUF):
        pltpu.async_copy(table.at[idx_sd.at[j, 0]], rows.at[j], gsem[j])

    def outer(i, carry):
        j0 = i * NBUF
        for b in range(NBUF):
            j = j0 + b
            pltpu.make_async_copy(table.at[idx_sd.at[j, 0]], rows.at[b],
                                  gsem[b]).wait()
            pltpu.sync_copy(rows.at[b], acc.at[idx_sd.at[j, 1]], add=True)

            @pl.when(j + NBUF < CT)
            def _():
                pltpu.async_copy(table.at[idx_sd.at[j + NBUF, 0]], rows.at[b],
                                 gsem[b])

        return carry

    lax.fori_loop(0, CT // NBUF, outer, 0)
    plsc.subcore_barrier()
    pltpu.sync_copy(acc.at[pl.ds(t * TS, TS)], s_out.at[c, pl.ds(t * TS, TS)])


_edge_call = functools.partial(
    pl.kernel,
    mesh=_MESH,
    out_type=jax.ShapeDtypeStruct((NC, R, DH), jnp.float32),
    compiler_params=_SC_PARAMS,
    scratch_types=[
        pltpu.VMEM((CT, 2, CHUNK), jnp.int32),
        pltpu.VMEM((NBUF, CHUNK, DH), jnp.float32),
        pltpu.VMEM_SHARED((R, DH), jnp.float32),
    ] + [pltpu.SemaphoreType.DMA] * NBUF,
)(_edge_body)


def _pack2(hd):
    h3 = jnp.reshape(hd, (BLK2, 2, D))
    p0 = jnp.concatenate([h3[:, 0, :DH], h3[:, 1, :DH]], axis=1)
    p1 = jnp.concatenate([h3[:, 0, DH:], h3[:, 1, DH:]], axis=1)
    return p0, p1


def _unpack2(p0, p1):
    u = jnp.concatenate([p0[:, :DH], p1[:, :DH]], axis=1)[:, None, :]
    v = jnp.concatenate([p0[:, DH:], p1[:, DH:]], axis=1)[:, None, :]
    return jnp.reshape(jnp.concatenate([u, v], axis=1), (BLK, D))


def _head_body(x_ref, w_ref, dinv_ref, o_ref):
    h = jnp.dot(x_ref[...], w_ref[...], preferred_element_type=jnp.float32)
    hd = h * dinv_ref[...]
    p0, p1 = _pack2(hd)
    o_ref[0] = p0
    o_ref[1] = p1


def _head(xp, W1, dinv):
    return pl.pallas_call(
        _head_body,
        grid=(NBLK,),
        in_specs=[
            pl.BlockSpec((BLK, D), lambda i: (i, 0)),
            pl.BlockSpec((D, D), lambda i: (0, 0)),
            pl.BlockSpec((BLK, 1), lambda i: (i, 0)),
        ],
        out_specs=pl.BlockSpec((NC, BLK2, D), lambda i: (0, i, 0)),
        out_shape=jax.ShapeDtypeStruct((NC, R2, D), jnp.float32),
    )(xp, W1, dinv)


def _mid_body(s_ref, hd_ref, dinv_ref, b_ref, w_ref, o_ref):
    t = _unpack2(s_ref[0] + hd_ref[0], s_ref[1] + hd_ref[1])
    a = jnp.maximum(t * dinv_ref[...] + b_ref[...], 0.0)
    hd2 = jnp.dot(a, w_ref[...],
                  preferred_element_type=jnp.float32) * dinv_ref[...]
    p0, p1 = _pack2(hd2)
    o_ref[0] = p0
    o_ref[1] = p1


def _mid(s, hd, dinv, b, W2):
    return pl.pallas_call(
        _mid_body,
        grid=(NBLK,),
        in_specs=[
            pl.BlockSpec((NC, BLK2, D), lambda i: (0, i, 0)),
            pl.BlockSpec((NC, BLK2, D), lambda i: (0, i, 0)),
            pl.BlockSpec((BLK, 1), lambda i: (i, 0)),
            pl.BlockSpec((1, D), lambda i: (0, 0)),
            pl.BlockSpec((D, D), lambda i: (0, 0)),
        ],
        out_specs=pl.BlockSpec((NC, BLK2, D), lambda i: (0, i, 0)),
        out_shape=jax.ShapeDtypeStruct((NC, R2, D), jnp.float32),
    )(s, hd, dinv, b, W2)


def _tail_body(s_ref, hd_ref, dinv_ref, b_ref, batch_ref, wl_ref, bl_ref,
               o_ref, sums, cnts):
    i = pl.program_id(0)

    @pl.when(i == 0)
    def _():
        sums[...] = jnp.zeros_like(sums)
        cnts[...] = jnp.zeros_like(cnts)

    t = _unpack2(s_ref[0] + hd_ref[0], s_ref[1] + hd_ref[1])
    h2 = jnp.maximum(t * dinv_ref[...] + b_ref[...], 0.0)
    onehot = (batch_ref[...] ==
              lax.broadcasted_iota(jnp.int32, (BLK, G), 1)).astype(jnp.float32)
    sums[...] += lax.dot_general(onehot, h2, (((0,), (0,)), ((), ())),
                                 preferred_element_type=jnp.float32)
    cnts[...] += lax.dot_general(onehot, jnp.ones((BLK, D), jnp.float32),
                                 (((0,), (0,)), ((), ())),
                                 preferred_element_type=jnp.float32)

    @pl.when(i == NBLK - 1)
    def _():
        pooled = sums[...] / jnp.maximum(cnts[...], 1.0)
        o_ref[...] = jnp.dot(pooled, wl_ref[...],
                             preferred_element_type=jnp.float32) + bl_ref[...]


def _tail(s, hd, dinv, b, batchp, Wlp, blp):
    return pl.pallas_call(
        _tail_body,
        grid=(NBLK,),
        in_specs=[
            pl.BlockSpec((NC, BLK2, D), lambda i: (0, i, 0)),
            pl.BlockSpec((NC, BLK2, D), lambda i: (0, i, 0)),
            pl.BlockSpec((BLK, 1), lambda i: (i, 0)),
            pl.BlockSpec((1, D), lambda i: (0, 0)),
            pl.BlockSpec((BLK, 1), lambda i: (i, 0)),
            pl.BlockSpec((D, 16), lambda i: (0, 0)),
            pl.BlockSpec((1, 16), lambda i: (0, 0)),
        ],
        out_specs=pl.BlockSpec((G, 16), lambda i: (0, 0)),
        out_shape=jax.ShapeDtypeStruct((G, 16), jnp.float32),
        scratch_shapes=[
            pltpu.VMEM((G, D), jnp.float32),
            pltpu.VMEM((G, D), jnp.float32),
        ],
    )(s, hd, dinv, b, batchp, Wlp, blp)


def kernel(x, edge_index, batch, W1, b1, W2, b2, Wl, bl):
    nchunk = E // CHUNK
    padc = NS * CT - nchunk
    ei = edge_index.astype(jnp.int32).reshape(2, nchunk, CHUNK)
    ei = jnp.transpose(ei, (1, 0, 2))
    pad_s = ((jnp.arange(padc * CHUNK, dtype=jnp.int32) * 997) % N).reshape(
        padc, 1, CHUNK)
    pad_d = (N + (jnp.arange(padc * CHUNK, dtype=jnp.int32) % (R - N))
             ).reshape(padc, 1, CHUNK)
    ei4 = jnp.concatenate([ei, jnp.concatenate([pad_s, pad_d], axis=1)],
                          axis=0).reshape(NS, CT, 2, CHUNK)

    xp = jnp.pad(x, ((0, R - N), (0, 0)))
    batchp = jnp.pad(batch.astype(jnp.int32), (0, R - N),
                     constant_values=G).reshape(R, 1)
    z16 = jnp.zeros((TS, 16), jnp.float32)
    zDH = jnp.zeros((TS, DH), jnp.float32)
    ones_rows = jnp.concatenate(
        [jnp.ones((CHUNK, 1), jnp.float32),
         jnp.zeros((CHUNK, 15), jnp.float32)], axis=1)
    Wlp = jnp.pad(Wl, ((0, 0), (0, 16 - NCLS)))
    blp = jnp.pad(bl, (0, 16 - NCLS)).reshape(1, 16)
    b1r = b1.reshape(1, D)
    b2r = b2.reshape(1, D)

    deg = _deg_call(ei4, z16, ones_rows)
    cnt = deg.reshape(NC, R)
    dinv = lax.rsqrt(cnt[0] + cnt[1] + 1.0).reshape(R, 1)

    hd1 = _head(xp, W1, dinv)
    s1 = _edge_call(hd1.reshape(NC, R, DH), ei4, zDH)
    hd2 = _mid(s1.reshape(NC, R2, D), hd1, dinv, b1r, W2)
    s2 = _edge_call(hd2.reshape(NC, R, DH), ei4, zDH)
    out16 = _tail(s2.reshape(NC, R2, D), hd2, dinv, b2r, batchp, Wlp, blp)
    return out16[:, :NCLS]
